# baseline (speedup 1.0000x reference)
import functools

import jax
import jax.numpy as jnp
from jax import lax
from jax.experimental import pallas as pl
from jax.experimental.pallas import tpu as pltpu
from jax.experimental.pallas import tpu_sc as plsc

L = 16
NS = 16
NC = 2


@functools.lru_cache(maxsize=None)
def _build(B: int, V: int):
    assert B % (NC * NS * L) == 0 and V % 8 == 0 and B % 128 == 0
    bpw = B // (NC * NS)

    mesh = plsc.VectorSubcoreMesh(
        core_axis_name="c", subcore_axis_name="s", num_cores=NC
    )

    @functools.partial(
        pl.kernel,
        out_type=jax.ShapeDtypeStruct((NC, L), jnp.float32),
        mesh=mesh,
        compiler_params=pltpu.CompilerParams(use_tc_tiling_on_sc=True),
        scratch_types=[
            pltpu.VMEM((bpw,), jnp.int32),
            pltpu.VMEM((bpw,), jnp.int32),
            pltpu.VMEM((bpw, 128), jnp.float32),
            pltpu.VMEM((L,), jnp.float32),
            pltpu.VMEM_SHARED((NS * L,), jnp.float32),
            pltpu.VMEM((NS * L,), jnp.float32),
            pltpu.SemaphoreType.DMA,
        ],
    )
    def gather_mean(lines, yref, out, y_v, idx_v, g_v, acc_v, shared, red_v, sem):
        sid = lax.axis_index("s")
        cid = lax.axis_index("c")
        wid = sid * NC + cid
        base = wid * bpw
        pltpu.sync_copy(yref.at[pl.ds(base, bpw)], y_v)
        iota = lax.iota(jnp.int32, L)
        bblock = lax.shift_left(lax.shift_right_logical(base, 7), 3)
        for c in range(bpw // L):
            yc = y_v[pl.ds(c * L, L)]
            line = (
                lax.shift_left(lax.shift_right_logical(yc, 3), 6)
                + bblock
                + lax.bitwise_and(yc, 7)
            )
            idx_v[pl.ds(c * L, L)] = line
        pltpu.async_copy(lines.at[idx_v], g_v, sem).wait()
        acc = jnp.zeros((L,), jnp.float32)
        for j in range(bpw):
            lane = lax.bitwise_and(base + j, 127)
            lane_c = lax.shift_left(lax.shift_right_logical(lane, 4), 4)
            chunk = g_v[j, pl.ds(lane_c, L)]
            acc = acc + jnp.where(iota == lax.bitwise_and(lane, L - 1), chunk, 0.0)
        acc_v[...] = acc
        pltpu.sync_copy(acc_v, shared.at[pl.ds(sid * L, L)])
        plsc.subcore_barrier()

        @pl.when(sid == 0)
        def _():
            pltpu.sync_copy(shared, red_v)
            tot = red_v[pl.ds(0, L)]
            for j in range(1, NS):
                tot = tot + red_v[pl.ds(j * L, L)]
            s = tot[0]
            for j in range(1, L):
                s = s + tot[j]
            acc_v[...] = jnp.broadcast_to(s * (1.0 / B), (L,))
            pltpu.sync_copy(acc_v, out.at[cid])

    return gather_mean


def kernel(output, y):
    B, V = output.shape
    lines = (
        output.T.reshape(V // 8, 8, B // 128, 128)
        .transpose(0, 2, 1, 3)
        .reshape((V * B) // 128, 128)
    )
    res = _build(B, V)(lines, y.astype(jnp.int32))
    return res[0, 0] + res[1, 0]


# speedup vs baseline: 1.2074x; 1.2074x over previous
import functools

import jax
import jax.numpy as jnp
from jax import lax
from jax.experimental import pallas as pl
from jax.experimental.pallas import tpu as pltpu
from jax.experimental.pallas import tpu_sc as plsc

L = 16
NS = 16


@functools.lru_cache(maxsize=None)
def _build(B: int, V: int):
    assert B % (NS * L) == 0 and V % 8 == 0 and B % 128 == 0
    bpw = B // NS

    mesh = plsc.VectorSubcoreMesh(
        core_axis_name="c", subcore_axis_name="s", num_cores=1
    )

    @functools.partial(
        pl.kernel,
        out_type=jax.ShapeDtypeStruct((L,), jnp.float32),
        mesh=mesh,
        compiler_params=pltpu.CompilerParams(use_tc_tiling_on_sc=True),
        scratch_types=[
            pltpu.VMEM((bpw,), jnp.int32),
            pltpu.VMEM((bpw,), jnp.int32),
            pltpu.VMEM((bpw, 128), jnp.float32),
            pltpu.VMEM((L,), jnp.float32),
            pltpu.VMEM_SHARED((NS * L,), jnp.float32),
            pltpu.VMEM((NS * L,), jnp.float32),
            pltpu.SemaphoreType.DMA,
        ],
    )
    def gather_mean(lines, yref, out, y_v, idx_v, g_v, acc_v, shared, red_v, sem):
        sid = lax.axis_index("s")
        base = sid * bpw
        pltpu.sync_copy(yref.at[pl.ds(base, bpw)], y_v)
        iota = lax.iota(jnp.int32, L)
        bblock = lax.shift_left(lax.shift_right_logical(base, 7), 3)
        copies = []
        for c in range(bpw // L):
            yc = y_v[pl.ds(c * L, L)]
            line = (
                lax.shift_left(lax.shift_right_logical(yc, 3), 6)
                + bblock
                + lax.bitwise_and(yc, 7)
            )
            idx_v[pl.ds(c * L, L)] = line
            copies.append(
                pltpu.async_copy(
                    lines.at[idx_v.at[pl.ds(c * L, L)]],
                    g_v.at[pl.ds(c * L, L)],
                    sem,
                )
            )
        acc = jnp.zeros((L,), jnp.float32)
        for c in range(bpw // L):
            copies[c].wait()
            for t in range(L):
                j = c * L + t
                lane = lax.bitwise_and(base + j, 127)
                lane_c = lax.shift_left(lax.shift_right_logical(lane, 4), 4)
                chunk = g_v[j, pl.ds(lane_c, L)]
                acc = acc + jnp.where(
                    iota == lax.bitwise_and(lane, L - 1), chunk, 0.0
                )
        acc_v[...] = acc
        pltpu.sync_copy(acc_v, shared.at[pl.ds(sid * L, L)])
        plsc.subcore_barrier()

        @pl.when(sid == 0)
        def _():
            pltpu.sync_copy(shared, red_v)
            tot = red_v[pl.ds(0, L)]
            for j in range(1, NS):
                tot = tot + red_v[pl.ds(j * L, L)]
            s = tot[0]
            for j in range(1, L):
                s = s + tot[j]
            acc_v[...] = jnp.broadcast_to(s * (1.0 / B), (L,))
            pltpu.sync_copy(acc_v, out)

    return gather_mean


def kernel(output, y):
    B, V = output.shape
    lines = (
        output.T.reshape(V // 8, 8, B // 128, 128)
        .transpose(0, 2, 1, 3)
        .reshape((V * B) // 128, 128)
    )
    res = _build(B, V)(lines, y.astype(jnp.int32))
    return res[0]


# speedup vs baseline: 1.2584x; 1.0423x over previous
import functools

import jax
import jax.numpy as jnp
from jax import lax
from jax.experimental import pallas as pl
from jax.experimental.pallas import tpu as pltpu
from jax.experimental.pallas import tpu_sc as plsc

L = 16
NS = 16


@functools.lru_cache(maxsize=None)
def _build(B: int, V: int):
    assert B % (NS * L) == 0 and V % 8 == 0 and B % 128 == 0
    bpw = B // NS

    mesh = plsc.VectorSubcoreMesh(
        core_axis_name="c", subcore_axis_name="s", num_cores=1
    )

    @functools.partial(
        pl.kernel,
        out_type=jax.ShapeDtypeStruct((L,), jnp.float32),
        mesh=mesh,
        compiler_params=pltpu.CompilerParams(use_tc_tiling_on_sc=True),
        scratch_types=[
            pltpu.VMEM((bpw,), jnp.int32),
            pltpu.VMEM((bpw,), jnp.int32),
            pltpu.VMEM((bpw,), jnp.float32),
            pltpu.VMEM((L,), jnp.float32),
            pltpu.VMEM_SHARED((NS * L,), jnp.float32),
            pltpu.VMEM((NS * L,), jnp.float32),
            pltpu.SemaphoreType.DMA,
        ],
    )
    def gather_mean(flat, yref, out, y_v, idx_v, g_v, acc_v, shared, red_v, sem):
        sid = lax.axis_index("s")
        base = sid * bpw
        pltpu.sync_copy(yref.at[pl.ds(base, bpw)], y_v)
        iota = lax.iota(jnp.int32, L)
        for c in range(bpw // L):
            yc = y_v[pl.ds(c * L, L)]
            b = base + c * L + iota
            word = (
                lax.shift_left(lax.shift_right_logical(yc, 3), 13)
                + lax.shift_left(lax.shift_right_logical(b, 7), 10)
                + lax.shift_left(lax.bitwise_and(yc, 7), 7)
                + lax.bitwise_and(b, 127)
            )
            idx_v[pl.ds(c * L, L)] = word
        pltpu.async_copy(flat.at[idx_v], g_v, sem).wait()
        acc = jnp.zeros((L,), jnp.float32)
        for c in range(bpw // L):
            acc = acc + g_v[pl.ds(c * L, L)]
        acc_v[...] = acc
        pltpu.sync_copy(acc_v, shared.at[pl.ds(sid * L, L)])
        plsc.subcore_barrier()

        @pl.when(sid == 0)
        def _():
            pltpu.sync_copy(shared, red_v)
            tot = red_v[pl.ds(0, L)]
            for j in range(1, NS):
                tot = tot + red_v[pl.ds(j * L, L)]
            s = tot[0]
            for j in range(1, L):
                s = s + tot[j]
            acc_v[...] = jnp.broadcast_to(s * (1.0 / B), (L,))
            pltpu.sync_copy(acc_v, out)

    return gather_mean


def kernel(output, y):
    B, V = output.shape
    flat = (
        output.T.reshape(V // 8, 8, B // 128, 128)
        .transpose(0, 2, 1, 3)
        .reshape(V * B)
    )
    res = _build(B, V)(flat, y.astype(jnp.int32))
    return res[0]
